# Initial kernel scaffold; baseline (speedup 1.0000x reference)
import jax
import jax.numpy as jnp
from jax.experimental import pallas as pl


def kernel(x, edge_index, b_l0_w, b_l0_b, b_l1_w, b_l1_b, b_res_w, b_res_b, m_l0_w, m_l0_b, m_l1_w, m_l1_b, m_res_w, m_res_b):
    raise NotImplementedError("write your pallas kernel here")


# speedup vs baseline: 8.7758x; 8.7758x over previous
import functools

import jax
import jax.numpy as jnp
from jax import lax
from jax.experimental import pallas as pl
from jax.experimental.pallas import tpu as pltpu
from jax.experimental.pallas import tpu_sc as plsc

N = 10000
E = 320000
D = 128
HOPS = 3
NC = 2
NS = 16
NW = NC * NS
K = 79
CPT = K * 128
EPAD = NW * CPT
NP = 10240
BLK = 512
GRID = NP // BLK
ROWS_PT = NP // NS
DUMMY = N

@functools.cache
def _sc_mesh():
    return plsc.VectorSubcoreMesh(core_axis_name="c", subcore_axis_name="s",
                                  num_cores=NC, num_subcores=NS)


def _deg_body(col_hbm, out_hbm, col_v, deg_v):
    c = lax.axis_index("c")
    s = lax.axis_index("s")
    w = c * NS + s
    pltpu.sync_copy(col_hbm.at[w], col_v)
    zeros16 = jnp.zeros((16,), jnp.float32)

    def zbody(j, carry):
        deg_v[pl.ds(j * 16, 16)] = zeros16
        return carry

    lax.fori_loop(0, NP // 16, zbody, 0)
    ones16 = jnp.ones((16,), jnp.float32)

    def body(j, carry):
        plsc.addupdate_scatter(deg_v, [col_v[j]], ones16)
        return carry

    lax.fori_loop(0, CPT // 16, body, 0)
    pltpu.sync_copy(deg_v, out_hbm.at[w])


_SC_PARAMS = pltpu.CompilerParams(needs_layout_passes=False)


@functools.cache
def _deg_kernel():
    return pl.kernel(
        _deg_body,
        out_type=jax.ShapeDtypeStruct((NW, NP), jnp.float32),
        mesh=_sc_mesh(),
        compiler_params=_SC_PARAMS,
        scratch_types=[
            pltpu.VMEM((CPT // 16, 16), jnp.int32),
            pltpu.VMEM((NP,), jnp.float32),
        ],
    )


def _run_deg(colp16):
    return _deg_kernel()(colp16)


def _hop_body(y_hbm, row_hbm, col_hbm, zeros_hbm, out_hbm,
              row_v, col_v, gbuf, acc, gsem):
    c = lax.axis_index("c")
    s = lax.axis_index("s")
    w = c * NS + s
    pltpu.sync_copy(row_hbm.at[w], row_v)
    pltpu.sync_copy(col_hbm.at[w], col_v)
    for jj in range(ROWS_PT // 128):
        pltpu.sync_copy(zeros_hbm, acc.at[pl.ds(s * ROWS_PT + jj * 128, 128)])
    plsc.subcore_barrier()

    def body(j, carry):
        pltpu.async_copy(y_hbm.at[row_v.at[j]], gbuf, gsem).wait()
        pltpu.sync_copy(gbuf, acc.at[col_v.at[j]], add=True)
        return carry

    lax.fori_loop(0, K, body, 0)
    plsc.subcore_barrier()
    pltpu.sync_copy(acc.at[pl.ds(s * ROWS_PT, ROWS_PT)],
                    out_hbm.at[c].at[pl.ds(s * ROWS_PT, ROWS_PT)])


@functools.cache
def _hop_kernel():
    return pl.kernel(
        _hop_body,
        out_type=jax.ShapeDtypeStruct((NC, NP, D), jnp.float32),
        mesh=_sc_mesh(),
        compiler_params=_SC_PARAMS,
        scratch_types=[
            pltpu.VMEM((K, 128), jnp.int32),
            pltpu.VMEM((K, 128), jnp.int32),
            pltpu.VMEM((128, D), jnp.float32),
            pltpu.VMEM_SHARED((NP, D), jnp.float32),
            pltpu.SemaphoreType.DMA,
        ],
    )


def _run_hop(y, rowp, colp, zeros128):
    return _hop_kernel()(y, rowp, colp, zeros128)


def _dg(a, w):
    return lax.dot_general(a, w, (((1,), (1,)), ((), ())),
                           preferred_element_type=jnp.float32,
                           precision=lax.Precision.HIGHEST)


def _scale_body(degp_ref, x_ref, y1_ref, dis_ref):
    degsum = jnp.sum(degp_ref[...], axis=0)[0]
    r = lax.broadcasted_iota(jnp.int32, (128, 128), 0)
    cc = lax.broadcasted_iota(jnp.int32, (128, 128), 1)
    eye = jnp.where(r == cc, 1.0, 0.0).astype(jnp.float32)
    ones = jnp.ones((128, 128), jnp.float32)
    parts = []
    for kk in range(BLK // 128):
        vk = degsum[kk][None, :]
        w = jnp.broadcast_to(vk, (128, 128)) * eye
        parts.append(lax.dot_general(w, ones, (((1,), (0,)), ((), ())),
                                     preferred_element_type=jnp.float32,
                                     precision=lax.Precision.HIGHEST))
    deg_blk = jnp.concatenate(parts, axis=0)
    dis_blk = lax.rsqrt(deg_blk)
    dis_ref[...] = dis_blk
    y1_ref[...] = dis_blk * x_ref[...]


def _scale_call(degp3, xp):
    return pl.pallas_call(
        _scale_body,
        grid=(GRID,),
        in_specs=[
            pl.BlockSpec((NW, 1, BLK // 128, 128), lambda i: (0, i, 0, 0)),
            pl.BlockSpec((BLK, D), lambda i: (i, 0)),
        ],
        out_specs=[
            pl.BlockSpec((BLK, D), lambda i: (i, 0)),
            pl.BlockSpec((BLK, D), lambda i: (i, 0)),
        ],
        out_shape=[
            jax.ShapeDtypeStruct((NP, D), jnp.float32),
            jax.ShapeDtypeStruct((NP, D), jnp.float32),
        ],
    )(degp3, xp)


def _mid_body(zp_ref, dis_ref, c1_ref, y2_ref):
    dis = dis_ref[...]
    c1 = dis * (zp_ref[0] + zp_ref[1])
    c1_ref[...] = c1
    y2_ref[...] = dis * c1


def _mid_call(z1p, dis):
    return pl.pallas_call(
        _mid_body,
        grid=(GRID,),
        in_specs=[
            pl.BlockSpec((NC, BLK, D), lambda i: (0, i, 0)),
            pl.BlockSpec((BLK, D), lambda i: (i, 0)),
        ],
        out_specs=[
            pl.BlockSpec((BLK, D), lambda i: (i, 0)),
            pl.BlockSpec((BLK, D), lambda i: (i, 0)),
        ],
        out_shape=[
            jax.ShapeDtypeStruct((NP, D), jnp.float32),
            jax.ShapeDtypeStruct((NP, D), jnp.float32),
        ],
    )(z1p, dis)


def _mlp_body(x_ref, c1_ref, z2p_ref, dis_ref,
              bl0w, bl0b, bl1w, bl1b, brw, brb,
              ml0w, ml0b, ml1w, ml1b, mrw, mrb, out_ref):
    dis = dis_ref[...]
    c2 = dis * (z2p_ref[0] + z2p_ref[1])

    def branch(v, i):
        res = _dg(v, brw[i]) + brb[i]
        h = jnp.maximum(_dg(v, bl0w[i]) + bl0b[i], 0.0)
        return _dg(h, bl1w[i]) + bl1b[i] + res

    h0 = branch(x_ref[...], 0)
    h1 = branch(c1_ref[...], 1)
    h2 = branch(c2, 2)
    h = jnp.concatenate([h0, h1, h2], axis=1)
    res = _dg(h, mrw[...]) + mrb[...]
    g = jnp.maximum(_dg(h, ml0w[...]) + ml0b[...], 0.0)
    out_ref[...] = _dg(g, ml1w[...]) + ml1b[...] + res


def _mlp_call(xp, c1, z2p, dis, bl0w, bl0b, bl1w, bl1b, brw, brb,
              ml0w, ml0b, ml1w, ml1b, mrw, mrb):
    full = lambda shape: pl.BlockSpec(shape, lambda i: tuple(0 for _ in shape))
    return pl.pallas_call(
        _mlp_body,
        grid=(GRID,),
        in_specs=[
            pl.BlockSpec((BLK, D), lambda i: (i, 0)),
            pl.BlockSpec((BLK, D), lambda i: (i, 0)),
            pl.BlockSpec((NC, BLK, D), lambda i: (0, i, 0)),
            pl.BlockSpec((BLK, D), lambda i: (i, 0)),
            full(bl0w.shape), full(bl0b.shape),
            full(bl1w.shape), full(bl1b.shape),
            full(brw.shape), full(brb.shape),
            full(ml0w.shape), full(ml0b.shape),
            full(ml1w.shape), full(ml1b.shape),
            full(mrw.shape), full(mrb.shape),
        ],
        out_specs=pl.BlockSpec((BLK, D), lambda i: (i, 0)),
        out_shape=jax.ShapeDtypeStruct((NP, D), jnp.float32),
    )(xp, c1, z2p, dis, bl0w, bl0b, bl1w, bl1b, brw, brb,
      ml0w, ml0b, ml1w, ml1b, mrw, mrb)


def kernel(x, edge_index, b_l0_w, b_l0_b, b_l1_w, b_l1_b, b_res_w, b_res_b,
           m_l0_w, m_l0_b, m_l1_w, m_l1_b, m_res_w, m_res_b):
    row = edge_index[0]
    col = edge_index[1]
    pad = EPAD - E
    rowp = jnp.concatenate([row, jnp.zeros((pad,), jnp.int32)]).reshape(NW, K, 128)
    colp_flat = jnp.concatenate([col, jnp.full((pad,), DUMMY, jnp.int32)])
    colp = colp_flat.reshape(NW, K, 128)
    colp16 = colp_flat.reshape(NW, CPT // 16, 16)
    xp = jnp.pad(x, ((0, NP - N), (0, 0)))
    zeros128 = jnp.zeros((128, D), jnp.float32)
    m_l0_b = m_l0_b.reshape(1, -1)
    m_l1_b = m_l1_b.reshape(1, -1)
    m_res_b = m_res_b.reshape(1, -1)

    degp = _run_deg(colp16)
    degp3 = degp.reshape(NW, GRID, BLK // 128, 128)

    y1, dis = _scale_call(degp3, xp)
    z1p = _run_hop(y1, rowp, colp, zeros128)
    c1, y2 = _mid_call(z1p, dis)
    z2p = _run_hop(y2, rowp, colp, zeros128)
    out = _mlp_call(xp, c1, z2p, dis,
                    b_l0_w, b_l0_b, b_l1_w, b_l1_b, b_res_w, b_res_b,
                    m_l0_w, m_l0_b, m_l1_w, m_l1_b, m_res_w, m_res_b)
    return out[:N]


# speedup vs baseline: 8.8188x; 1.0049x over previous
import functools

import jax
import jax.numpy as jnp
from jax import lax
from jax.experimental import pallas as pl
from jax.experimental.pallas import tpu as pltpu
from jax.experimental.pallas import tpu_sc as plsc

N = 10000
E = 320000
D = 128
HOPS = 3
NC = 2
NS = 16
NW = NC * NS
K = 79
CPT = K * 128
EPAD = NW * CPT
NP = 10240
BLK = 512
GRID = NP // BLK
ROWS_PT = NP // NS
DUMMY = N

@functools.cache
def _sc_mesh():
    return plsc.VectorSubcoreMesh(core_axis_name="c", subcore_axis_name="s",
                                  num_cores=NC, num_subcores=NS)


def _deg_body(col_hbm, out_hbm, col_v, deg_v):
    c = lax.axis_index("c")
    s = lax.axis_index("s")
    w = c * NS + s
    pltpu.sync_copy(col_hbm.at[w], col_v)
    zeros16 = jnp.zeros((16,), jnp.float32)

    def zbody(j, carry):
        deg_v[pl.ds(j * 16, 16)] = zeros16
        return carry

    lax.fori_loop(0, NP // 16, zbody, 0)
    ones16 = jnp.ones((16,), jnp.float32)

    def body(j, carry):
        plsc.addupdate_scatter(deg_v, [col_v[j]], ones16)
        return carry

    lax.fori_loop(0, CPT // 16, body, 0)
    pltpu.sync_copy(deg_v, out_hbm.at[w])


_SC_PARAMS = pltpu.CompilerParams(needs_layout_passes=False)


@functools.cache
def _deg_kernel():
    return pl.kernel(
        _deg_body,
        out_type=jax.ShapeDtypeStruct((NW, NP), jnp.float32),
        mesh=_sc_mesh(),
        compiler_params=_SC_PARAMS,
        scratch_types=[
            pltpu.VMEM((CPT // 16, 16), jnp.int32),
            pltpu.VMEM((NP,), jnp.float32),
        ],
    )


def _run_deg(colp16):
    return _deg_kernel()(colp16)


def _hop_body(y_hbm, row_hbm, col_hbm, zeros_hbm, out_hbm,
              row_v, col_v, gbuf, acc, gsem):
    c = lax.axis_index("c")
    s = lax.axis_index("s")
    w = c * NS + s
    pltpu.sync_copy(row_hbm.at[w], row_v)
    pltpu.sync_copy(col_hbm.at[w], col_v)
    for jj in range(ROWS_PT // 128):
        pltpu.sync_copy(zeros_hbm, acc.at[pl.ds(s * ROWS_PT + jj * 128, 128)])
    plsc.subcore_barrier()

    def body(j, carry):
        pltpu.async_copy(y_hbm.at[row_v.at[j]], gbuf, gsem).wait()
        pltpu.sync_copy(gbuf, acc.at[col_v.at[j]], add=True)
        return carry

    lax.fori_loop(0, K, body, 0)
    plsc.subcore_barrier()
    pltpu.sync_copy(acc.at[pl.ds(s * ROWS_PT, ROWS_PT)],
                    out_hbm.at[c].at[pl.ds(s * ROWS_PT, ROWS_PT)])


@functools.cache
def _hop_kernel():
    return pl.kernel(
        _hop_body,
        out_type=jax.ShapeDtypeStruct((NC, NP, D), jnp.float32),
        mesh=_sc_mesh(),
        compiler_params=_SC_PARAMS,
        scratch_types=[
            pltpu.VMEM((K, 128), jnp.int32),
            pltpu.VMEM((K, 128), jnp.int32),
            pltpu.VMEM((128, D), jnp.float32),
            pltpu.VMEM_SHARED((NP, D), jnp.float32),
            pltpu.SemaphoreType.DMA,
        ],
    )


def _run_hop(y, rowp, colp, zeros128):
    return _hop_kernel()(y, rowp, colp, zeros128)


def _dg(a, w):
    return lax.dot_general(a, w, (((1,), (1,)), ((), ())),
                           preferred_element_type=jnp.float32,
                           precision=lax.Precision.HIGHEST)


def _scale_body(degp_ref, x_ref, y1_ref, dis_ref):
    degsum = jnp.sum(degp_ref[...], axis=0)[0]
    r = lax.broadcasted_iota(jnp.int32, (128, 128), 0)
    cc = lax.broadcasted_iota(jnp.int32, (128, 128), 1)
    eye = jnp.where(r == cc, 1.0, 0.0).astype(jnp.float32)
    ones = jnp.ones((128, 128), jnp.float32)
    parts = []
    for kk in range(BLK // 128):
        vk = degsum[kk][None, :]
        w = jnp.broadcast_to(vk, (128, 128)) * eye
        parts.append(lax.dot_general(w, ones, (((1,), (0,)), ((), ())),
                                     preferred_element_type=jnp.float32,
                                     precision=lax.Precision.HIGHEST))
    deg_blk = jnp.concatenate(parts, axis=0)
    dis_blk = lax.rsqrt(deg_blk)
    dis_ref[...] = dis_blk
    y1_ref[...] = dis_blk * x_ref[...]


def _scale_call(degp3, xp):
    return pl.pallas_call(
        _scale_body,
        grid=(GRID,),
        in_specs=[
            pl.BlockSpec((NW, 1, BLK // 128, 128), lambda i: (0, i, 0, 0)),
            pl.BlockSpec((BLK, D), lambda i: (i, 0)),
        ],
        out_specs=[
            pl.BlockSpec((BLK, D), lambda i: (i, 0)),
            pl.BlockSpec((BLK, D), lambda i: (i, 0)),
        ],
        out_shape=[
            jax.ShapeDtypeStruct((NP, D), jnp.float32),
            jax.ShapeDtypeStruct((NP, D), jnp.float32),
        ],
    )(degp3, xp)


def _mid_body(zp_ref, dis_ref, c1_ref, y2_ref):
    dis = dis_ref[...]
    c1 = dis * (zp_ref[0] + zp_ref[1])
    c1_ref[...] = c1
    y2_ref[...] = dis * c1


def _mid_call(z1p, dis):
    return pl.pallas_call(
        _mid_body,
        grid=(GRID,),
        in_specs=[
            pl.BlockSpec((NC, BLK, D), lambda i: (0, i, 0)),
            pl.BlockSpec((BLK, D), lambda i: (i, 0)),
        ],
        out_specs=[
            pl.BlockSpec((BLK, D), lambda i: (i, 0)),
            pl.BlockSpec((BLK, D), lambda i: (i, 0)),
        ],
        out_shape=[
            jax.ShapeDtypeStruct((NP, D), jnp.float32),
            jax.ShapeDtypeStruct((NP, D), jnp.float32),
        ],
    )(z1p, dis)


def _mlp_body(x_ref, c1_ref, z2p_ref, dis_ref,
              bl0w, bl0b, bl1w, bl1b, brw, brb,
              ml0w, ml0b, ml1w, ml1b, mrw, mrb, out_ref):
    dis = dis_ref[...]
    c2 = dis * (z2p_ref[0] + z2p_ref[1])

    def branch(v, i):
        res = _dg(v, brw[i]) + brb[i]
        h = jnp.maximum(_dg(v, bl0w[i]) + bl0b[i], 0.0)
        return _dg(h, bl1w[i]) + bl1b[i] + res

    h0 = branch(x_ref[...], 0)
    h1 = branch(c1_ref[...], 1)
    h2 = branch(c2, 2)
    h = jnp.concatenate([h0, h1, h2], axis=1)
    res = _dg(h, mrw[...]) + mrb[...]
    g = jnp.maximum(_dg(h, ml0w[...]) + ml0b[...], 0.0)
    out_ref[...] = _dg(g, ml1w[...]) + ml1b[...] + res


def _mlp_call(xp, c1, z2p, dis, bl0w, bl0b, bl1w, bl1b, brw, brb,
              ml0w, ml0b, ml1w, ml1b, mrw, mrb):
    full = lambda shape: pl.BlockSpec(shape, lambda i: tuple(0 for _ in shape))
    return pl.pallas_call(
        _mlp_body,
        grid=(GRID,),
        in_specs=[
            pl.BlockSpec((BLK, D), lambda i: (i, 0)),
            pl.BlockSpec((BLK, D), lambda i: (i, 0)),
            pl.BlockSpec((NC, BLK, D), lambda i: (0, i, 0)),
            pl.BlockSpec((BLK, D), lambda i: (i, 0)),
            full(bl0w.shape), full(bl0b.shape),
            full(bl1w.shape), full(bl1b.shape),
            full(brw.shape), full(brb.shape),
            full(ml0w.shape), full(ml0b.shape),
            full(ml1w.shape), full(ml1b.shape),
            full(mrw.shape), full(mrb.shape),
        ],
        out_specs=pl.BlockSpec((BLK, D), lambda i: (i, 0)),
        out_shape=jax.ShapeDtypeStruct((NP, D), jnp.float32),
    )(xp, c1, z2p, dis, bl0w, bl0b, bl1w, bl1b, brw, brb,
      ml0w, ml0b, ml1w, ml1b, mrw, mrb)


def kernel(x, edge_index, b_l0_w, b_l0_b, b_l1_w, b_l1_b, b_res_w, b_res_b,
           m_l0_w, m_l0_b, m_l1_w, m_l1_b, m_res_w, m_res_b):
    row = edge_index[0]
    col = edge_index[1]
    pad = EPAD - E
    rowp = jnp.concatenate([row, jnp.zeros((pad,), jnp.int32)]).reshape(NW, K, 128)
    pad_dst = DUMMY + jnp.arange(pad, dtype=jnp.int32) % (NP - N)
    colp_flat = jnp.concatenate([col, pad_dst])
    colp = colp_flat.reshape(NW, K, 128)
    colp16 = colp_flat.reshape(NW, CPT // 16, 16)
    xp = jnp.pad(x, ((0, NP - N), (0, 0)))
    zeros128 = jnp.zeros((128, D), jnp.float32)
    m_l0_b = m_l0_b.reshape(1, -1)
    m_l1_b = m_l1_b.reshape(1, -1)
    m_res_b = m_res_b.reshape(1, -1)

    degp = _run_deg(colp16)
    degp3 = degp.reshape(NW, GRID, BLK // 128, 128)

    y1, dis = _scale_call(degp3, xp)
    z1p = _run_hop(y1, rowp, colp, zeros128)
    c1, y2 = _mid_call(z1p, dis)
    z2p = _run_hop(y2, rowp, colp, zeros128)
    out = _mlp_call(xp, c1, z2p, dis,
                    b_l0_w, b_l0_b, b_l1_w, b_l1_b, b_res_w, b_res_b,
                    m_l0_w, m_l0_b, m_l1_w, m_l1_b, m_res_w, m_res_b)
    return out[:N]


# speedup vs baseline: 12.3568x; 1.4012x over previous
import functools

import jax
import jax.numpy as jnp
from jax import lax
from jax.experimental import pallas as pl
from jax.experimental.pallas import tpu as pltpu
from jax.experimental.pallas import tpu_sc as plsc

N = 10000
E = 320000
D = 128
HOPS = 3
NC = 2
NS = 16
NW = NC * NS
K = 79
CPT = K * 128
EPAD = NW * CPT
K0 = 104
K1 = 53
E0 = NS * K0 * 128
E1 = NS * K1 * 128
NP = 10240
BLK = 512
GRID = NP // BLK
ROWS_PT = NP // NS
DUMMY = N

@functools.cache
def _sc_mesh():
    return plsc.VectorSubcoreMesh(core_axis_name="c", subcore_axis_name="s",
                                  num_cores=NC, num_subcores=NS)


def _deg_body(col_hbm, out_hbm, col_v, deg_v):
    c = lax.axis_index("c")
    s = lax.axis_index("s")
    w = c * NS + s
    pltpu.sync_copy(col_hbm.at[w], col_v)
    zeros16 = jnp.zeros((16,), jnp.float32)

    def zbody(j, carry):
        deg_v[pl.ds(j * 16, 16)] = zeros16
        return carry

    lax.fori_loop(0, NP // 16, zbody, 0)
    ones16 = jnp.ones((16,), jnp.float32)

    def body(j, carry):
        plsc.addupdate_scatter(deg_v, [col_v[j]], ones16)
        return carry

    lax.fori_loop(0, CPT // 16, body, 0)
    pltpu.sync_copy(deg_v, out_hbm.at[w])


_SC_PARAMS = pltpu.CompilerParams(needs_layout_passes=False)


@functools.cache
def _deg_kernel():
    return pl.kernel(
        _deg_body,
        out_type=jax.ShapeDtypeStruct((NW, NP), jnp.float32),
        mesh=_sc_mesh(),
        compiler_params=_SC_PARAMS,
        scratch_types=[
            pltpu.VMEM((CPT // 16, 16), jnp.int32),
            pltpu.VMEM((NP,), jnp.float32),
        ],
    )


def _run_deg(colp16):
    return _deg_kernel()(colp16)


def _hop_body(y_hbm, row0_hbm, col0_hbm, row1_hbm, col1_hbm, zeros_hbm,
              out_hbm, row_v, col_v, gbuf, acc, gsem):
    c = lax.axis_index("c")
    s = lax.axis_index("s")

    @pl.when(c == 0)
    def _():
        pltpu.sync_copy(row0_hbm.at[s], row_v.at[pl.ds(0, K0)])
        pltpu.sync_copy(col0_hbm.at[s], col_v.at[pl.ds(0, K0)])

    @pl.when(c == 1)
    def _():
        pltpu.sync_copy(row1_hbm.at[s], row_v.at[pl.ds(0, K1)])
        pltpu.sync_copy(col1_hbm.at[s], col_v.at[pl.ds(0, K1)])

    for jj in range(ROWS_PT // 128):
        pltpu.sync_copy(zeros_hbm, acc.at[pl.ds(s * ROWS_PT + jj * 128, 128)])
    plsc.subcore_barrier()
    nk = jnp.where(c == 0, K0, K1)

    def body(j, carry):
        pltpu.async_copy(y_hbm.at[row_v.at[j]], gbuf, gsem).wait()
        pltpu.sync_copy(gbuf, acc.at[col_v.at[j]], add=True)
        return carry

    lax.fori_loop(0, nk, body, 0)
    plsc.subcore_barrier()
    pltpu.sync_copy(acc.at[pl.ds(s * ROWS_PT, ROWS_PT)],
                    out_hbm.at[c].at[pl.ds(s * ROWS_PT, ROWS_PT)])


@functools.cache
def _hop_kernel():
    return pl.kernel(
        _hop_body,
        out_type=jax.ShapeDtypeStruct((NC, NP, D), jnp.float32),
        mesh=_sc_mesh(),
        compiler_params=_SC_PARAMS,
        scratch_types=[
            pltpu.VMEM((K0, 128), jnp.int32),
            pltpu.VMEM((K0, 128), jnp.int32),
            pltpu.VMEM((128, D), jnp.float32),
            pltpu.VMEM_SHARED((NP, D), jnp.float32),
            pltpu.SemaphoreType.DMA,
        ],
    )


def _run_hop(y, row0, col0, row1, col1, zeros128):
    return _hop_kernel()(y, row0, col0, row1, col1, zeros128)


def _dg(a, w):
    return lax.dot_general(a, w, (((1,), (1,)), ((), ())),
                           preferred_element_type=jnp.float32,
                           precision=lax.Precision.HIGHEST)


def _scale_body(degp_ref, x_ref, y1_ref, dis_ref):
    degsum = jnp.sum(degp_ref[...], axis=0)[0]
    r = lax.broadcasted_iota(jnp.int32, (128, 128), 0)
    cc = lax.broadcasted_iota(jnp.int32, (128, 128), 1)
    eye = jnp.where(r == cc, 1.0, 0.0).astype(jnp.float32)
    ones = jnp.ones((128, 128), jnp.float32)
    parts = []
    for kk in range(BLK // 128):
        vk = degsum[kk][None, :]
        w = jnp.broadcast_to(vk, (128, 128)) * eye
        parts.append(lax.dot_general(w, ones, (((1,), (0,)), ((), ())),
                                     preferred_element_type=jnp.float32,
                                     precision=lax.Precision.HIGHEST))
    deg_blk = jnp.concatenate(parts, axis=0)
    dis_blk = lax.rsqrt(deg_blk)
    dis_ref[...] = dis_blk
    y1_ref[...] = dis_blk * x_ref[...]


def _scale_call(degp3, xp):
    return pl.pallas_call(
        _scale_body,
        grid=(GRID,),
        in_specs=[
            pl.BlockSpec((NW, 1, BLK // 128, 128), lambda i: (0, i, 0, 0)),
            pl.BlockSpec((BLK, D), lambda i: (i, 0)),
        ],
        out_specs=[
            pl.BlockSpec((BLK, D), lambda i: (i, 0)),
            pl.BlockSpec((BLK, D), lambda i: (i, 0)),
        ],
        out_shape=[
            jax.ShapeDtypeStruct((NP, D), jnp.float32),
            jax.ShapeDtypeStruct((NP, D), jnp.float32),
        ],
    )(degp3, xp)


def _mid_body(zp_ref, dis_ref, c1_ref, y2_ref):
    dis = dis_ref[...]
    c1 = dis * (zp_ref[0] + zp_ref[1])
    c1_ref[...] = c1
    y2_ref[...] = dis * c1


def _mid_call(z1p, dis):
    return pl.pallas_call(
        _mid_body,
        grid=(GRID,),
        in_specs=[
            pl.BlockSpec((NC, BLK, D), lambda i: (0, i, 0)),
            pl.BlockSpec((BLK, D), lambda i: (i, 0)),
        ],
        out_specs=[
            pl.BlockSpec((BLK, D), lambda i: (i, 0)),
            pl.BlockSpec((BLK, D), lambda i: (i, 0)),
        ],
        out_shape=[
            jax.ShapeDtypeStruct((NP, D), jnp.float32),
            jax.ShapeDtypeStruct((NP, D), jnp.float32),
        ],
    )(z1p, dis)


def _mlp_body(x_ref, c1_ref, z2p_ref, dis_ref,
              bl0w, bl0b, bl1w, bl1b, brw, brb,
              ml0w, ml0b, ml1w, ml1b, mrw, mrb, out_ref):
    dis = dis_ref[...]
    c2 = dis * (z2p_ref[0] + z2p_ref[1])

    def branch(v, i):
        res = _dg(v, brw[i]) + brb[i]
        h = jnp.maximum(_dg(v, bl0w[i]) + bl0b[i], 0.0)
        return _dg(h, bl1w[i]) + bl1b[i] + res

    h0 = branch(x_ref[...], 0)
    h1 = branch(c1_ref[...], 1)
    h2 = branch(c2, 2)
    h = jnp.concatenate([h0, h1, h2], axis=1)
    res = _dg(h, mrw[...]) + mrb[...]
    g = jnp.maximum(_dg(h, ml0w[...]) + ml0b[...], 0.0)
    out_ref[...] = _dg(g, ml1w[...]) + ml1b[...] + res


def _mlp_call(xp, c1, z2p, dis, bl0w, bl0b, bl1w, bl1b, brw, brb,
              ml0w, ml0b, ml1w, ml1b, mrw, mrb):
    full = lambda shape: pl.BlockSpec(shape, lambda i: tuple(0 for _ in shape))
    return pl.pallas_call(
        _mlp_body,
        grid=(GRID,),
        in_specs=[
            pl.BlockSpec((BLK, D), lambda i: (i, 0)),
            pl.BlockSpec((BLK, D), lambda i: (i, 0)),
            pl.BlockSpec((NC, BLK, D), lambda i: (0, i, 0)),
            pl.BlockSpec((BLK, D), lambda i: (i, 0)),
            full(bl0w.shape), full(bl0b.shape),
            full(bl1w.shape), full(bl1b.shape),
            full(brw.shape), full(brb.shape),
            full(ml0w.shape), full(ml0b.shape),
            full(ml1w.shape), full(ml1b.shape),
            full(mrw.shape), full(mrb.shape),
        ],
        out_specs=pl.BlockSpec((BLK, D), lambda i: (i, 0)),
        out_shape=jax.ShapeDtypeStruct((NP, D), jnp.float32),
    )(xp, c1, z2p, dis, bl0w, bl0b, bl1w, bl1b, brw, brb,
      ml0w, ml0b, ml1w, ml1b, mrw, mrb)


def kernel(x, edge_index, b_l0_w, b_l0_b, b_l1_w, b_l1_b, b_res_w, b_res_b,
           m_l0_w, m_l0_b, m_l1_w, m_l1_b, m_res_w, m_res_b):
    row = edge_index[0]
    col = edge_index[1]
    pad = EPAD - E
    pad_dst = DUMMY + jnp.arange(pad, dtype=jnp.int32) % (NP - N)
    colp16 = jnp.concatenate([col, pad_dst]).reshape(NW, CPT // 16, 16)
    pad_h = E0 + E1 - E
    pad_dst_h = DUMMY + jnp.arange(pad_h, dtype=jnp.int32) % (NP - N)
    row_h = jnp.concatenate([row, jnp.zeros((pad_h,), jnp.int32)])
    col_h = jnp.concatenate([col, pad_dst_h])
    row0 = row_h[:E0].reshape(NS, K0, 128)
    col0 = col_h[:E0].reshape(NS, K0, 128)
    row1 = row_h[E0:].reshape(NS, K1, 128)
    col1 = col_h[E0:].reshape(NS, K1, 128)
    xp = jnp.pad(x, ((0, NP - N), (0, 0)))
    zeros128 = jnp.zeros((128, D), jnp.float32)
    m_l0_b = m_l0_b.reshape(1, -1)
    m_l1_b = m_l1_b.reshape(1, -1)
    m_res_b = m_res_b.reshape(1, -1)

    degp = _run_deg(colp16)
    degp3 = degp.reshape(NW, GRID, BLK // 128, 128)

    y1, dis = _scale_call(degp3, xp)
    z1p = _run_hop(y1, row0, col0, row1, col1, zeros128)
    c1, y2 = _mid_call(z1p, dis)
    z2p = _run_hop(y2, row0, col0, row1, col1, zeros128)
    out = _mlp_call(xp, c1, z2p, dis,
                    b_l0_w, b_l0_b, b_l1_w, b_l1_b, b_res_w, b_res_b,
                    m_l0_w, m_l0_b, m_l1_w, m_l1_b, m_res_w, m_res_b)
    return out[:N]


# speedup vs baseline: 14.9886x; 1.2130x over previous
import functools

import jax
import jax.numpy as jnp
from jax import lax
from jax.experimental import pallas as pl
from jax.experimental.pallas import tpu as pltpu
from jax.experimental.pallas import tpu_sc as plsc

N = 10000
E = 320000
D = 128
HOPS = 3
NC = 2
NS = 16
NW = NC * NS
K = 79
CPT = K * 128
EPAD = NW * CPT
CH = 64
M0 = 176
M1 = 138
E0 = NS * M0 * CH
E1 = NS * M1 * CH
NACC = 10016
RA = 632
RA_LAST = NACC - 15 * RA
NP = 10240
BLK = 512
GRID = NP // BLK
ROWS_PT = NP // NS
DUMMY = N

@functools.cache
def _sc_mesh():
    return plsc.VectorSubcoreMesh(core_axis_name="c", subcore_axis_name="s",
                                  num_cores=NC, num_subcores=NS)


def _deg_body(col_hbm, out_hbm, col_v, deg_v):
    c = lax.axis_index("c")
    s = lax.axis_index("s")
    w = c * NS + s
    pltpu.sync_copy(col_hbm.at[w], col_v)
    zeros16 = jnp.zeros((16,), jnp.float32)

    def zbody(j, carry):
        deg_v[pl.ds(j * 16, 16)] = zeros16
        return carry

    lax.fori_loop(0, NP // 16, zbody, 0)
    ones16 = jnp.ones((16,), jnp.float32)

    def body(j, carry):
        plsc.addupdate_scatter(deg_v, [col_v[j]], ones16)
        return carry

    lax.fori_loop(0, CPT // 16, body, 0)
    pltpu.sync_copy(deg_v, out_hbm.at[w])


_SC_PARAMS = pltpu.CompilerParams(needs_layout_passes=False)


@functools.cache
def _deg_kernel():
    return pl.kernel(
        _deg_body,
        out_type=jax.ShapeDtypeStruct((NW, NP), jnp.float32),
        mesh=_sc_mesh(),
        compiler_params=_SC_PARAMS,
        scratch_types=[
            pltpu.VMEM((CPT // 16, 16), jnp.int32),
            pltpu.VMEM((NP,), jnp.float32),
        ],
    )


def _run_deg(colp16):
    return _deg_kernel()(colp16)


def _hop_body(y_hbm, row0_hbm, col0_hbm, row1_hbm, col1_hbm, zeros_hbm,
              out_hbm, row_v, col_v, gbuf0, gbuf1, acc, g0, g1, s0, s1):
    c = lax.axis_index("c")
    s = lax.axis_index("s")

    @pl.when(c == 0)
    def _():
        pltpu.sync_copy(row0_hbm.at[s], row_v.at[pl.ds(0, M0 * CH)])
        pltpu.sync_copy(col0_hbm.at[s], col_v.at[pl.ds(0, M0)])

    @pl.when(c == 1)
    def _():
        pltpu.sync_copy(row1_hbm.at[s], row_v.at[pl.ds(0, M1 * CH)])
        pltpu.sync_copy(col1_hbm.at[s], col_v.at[pl.ds(0, M1)])

    base = s * RA
    for jj in range(4):
        pltpu.sync_copy(zeros_hbm, acc.at[pl.ds(base + jj * 128, 128)])

    @pl.when(s < NS - 1)
    def _():
        pltpu.sync_copy(zeros_hbm.at[pl.ds(0, RA - 512)],
                        acc.at[pl.ds(base + 512, RA - 512)])

    @pl.when(s == NS - 1)
    def _():
        pltpu.sync_copy(zeros_hbm.at[pl.ds(0, RA_LAST - 512)],
                        acc.at[pl.ds(base + 512, RA_LAST - 512)])

    plsc.subcore_barrier()
    n = jnp.where(c == 0, M0, M1)

    def gather(j, buf, sem):
        return pltpu.async_copy(y_hbm.at[row_v.at[pl.ds(j * CH, CH)]],
                                buf, sem)

    gather(0, gbuf0, g0)
    gather(1, gbuf1, g1)

    def body(i, carry):
        j = 2 * i
        pltpu.make_async_copy(y_hbm.at[row_v.at[pl.ds(j * CH, CH)]],
                              gbuf0, g0).wait()
        pltpu.async_copy(gbuf0, acc.at[col_v.at[j]], s0, add=True).wait()

        @pl.when(j + 2 < n)
        def _():
            gather(j + 2, gbuf0, g0)

        pltpu.make_async_copy(y_hbm.at[row_v.at[pl.ds((j + 1) * CH, CH)]],
                              gbuf1, g1).wait()
        pltpu.async_copy(gbuf1, acc.at[col_v.at[j + 1]], s1, add=True).wait()

        @pl.when(j + 3 < n)
        def _():
            gather(j + 3, gbuf1, g1)

        return carry

    lax.fori_loop(0, n // 2, body, 0)
    plsc.subcore_barrier()

    @pl.when(s < NS - 1)
    def _():
        pltpu.sync_copy(acc.at[pl.ds(base, RA)],
                        out_hbm.at[c].at[pl.ds(base, RA)])

    @pl.when(s == NS - 1)
    def _():
        pltpu.sync_copy(acc.at[pl.ds(base, RA_LAST)],
                        out_hbm.at[c].at[pl.ds(base, RA_LAST)])


@functools.cache
def _hop_kernel():
    return pl.kernel(
        _hop_body,
        out_type=jax.ShapeDtypeStruct((NC, NP, D), jnp.float32),
        mesh=_sc_mesh(),
        compiler_params=_SC_PARAMS,
        scratch_types=[
            pltpu.VMEM((M0 * CH,), jnp.int32),
            pltpu.VMEM((M0, CH), jnp.int32),
            pltpu.VMEM((CH, D), jnp.float32),
            pltpu.VMEM((CH, D), jnp.float32),
            pltpu.VMEM_SHARED((NACC, D), jnp.float32),
            pltpu.SemaphoreType.DMA,
            pltpu.SemaphoreType.DMA,
            pltpu.SemaphoreType.DMA,
            pltpu.SemaphoreType.DMA,
        ],
    )


def _run_hop(y, row0, col0, row1, col1, zeros128):
    return _hop_kernel()(y, row0, col0, row1, col1, zeros128)


def _dg(a, w):
    return lax.dot_general(a, w, (((1,), (1,)), ((), ())),
                           preferred_element_type=jnp.float32,
                           precision=lax.Precision.HIGHEST)


def _scale_body(degp_ref, x_ref, y1_ref, dis_ref):
    degsum = jnp.sum(degp_ref[...], axis=0)[0]
    r = lax.broadcasted_iota(jnp.int32, (128, 128), 0)
    cc = lax.broadcasted_iota(jnp.int32, (128, 128), 1)
    eye = jnp.where(r == cc, 1.0, 0.0).astype(jnp.float32)
    ones = jnp.ones((128, 128), jnp.float32)
    parts = []
    for kk in range(BLK // 128):
        vk = degsum[kk][None, :]
        w = jnp.broadcast_to(vk, (128, 128)) * eye
        parts.append(lax.dot_general(w, ones, (((1,), (0,)), ((), ())),
                                     preferred_element_type=jnp.float32,
                                     precision=lax.Precision.HIGHEST))
    deg_blk = jnp.concatenate(parts, axis=0)
    dis_blk = lax.rsqrt(deg_blk)
    dis_ref[...] = dis_blk
    y1_ref[...] = dis_blk * x_ref[...]


def _scale_call(degp3, xp):
    return pl.pallas_call(
        _scale_body,
        grid=(GRID,),
        in_specs=[
            pl.BlockSpec((NW, 1, BLK // 128, 128), lambda i: (0, i, 0, 0)),
            pl.BlockSpec((BLK, D), lambda i: (i, 0)),
        ],
        out_specs=[
            pl.BlockSpec((BLK, D), lambda i: (i, 0)),
            pl.BlockSpec((BLK, D), lambda i: (i, 0)),
        ],
        out_shape=[
            jax.ShapeDtypeStruct((NP, D), jnp.float32),
            jax.ShapeDtypeStruct((NP, D), jnp.float32),
        ],
    )(degp3, xp)


def _mid_body(zp_ref, dis_ref, c1_ref, y2_ref):
    dis = dis_ref[...]
    c1 = dis * (zp_ref[0] + zp_ref[1])
    c1_ref[...] = c1
    y2_ref[...] = dis * c1


def _mid_call(z1p, dis):
    return pl.pallas_call(
        _mid_body,
        grid=(GRID,),
        in_specs=[
            pl.BlockSpec((NC, BLK, D), lambda i: (0, i, 0)),
            pl.BlockSpec((BLK, D), lambda i: (i, 0)),
        ],
        out_specs=[
            pl.BlockSpec((BLK, D), lambda i: (i, 0)),
            pl.BlockSpec((BLK, D), lambda i: (i, 0)),
        ],
        out_shape=[
            jax.ShapeDtypeStruct((NP, D), jnp.float32),
            jax.ShapeDtypeStruct((NP, D), jnp.float32),
        ],
    )(z1p, dis)


def _mlp_body(x_ref, c1_ref, z2p_ref, dis_ref,
              bl0w, bl0b, bl1w, bl1b, brw, brb,
              ml0w, ml0b, ml1w, ml1b, mrw, mrb, out_ref):
    dis = dis_ref[...]
    c2 = dis * (z2p_ref[0] + z2p_ref[1])

    def branch(v, i):
        res = _dg(v, brw[i]) + brb[i]
        h = jnp.maximum(_dg(v, bl0w[i]) + bl0b[i], 0.0)
        return _dg(h, bl1w[i]) + bl1b[i] + res

    h0 = branch(x_ref[...], 0)
    h1 = branch(c1_ref[...], 1)
    h2 = branch(c2, 2)
    h = jnp.concatenate([h0, h1, h2], axis=1)
    res = _dg(h, mrw[...]) + mrb[...]
    g = jnp.maximum(_dg(h, ml0w[...]) + ml0b[...], 0.0)
    out_ref[...] = _dg(g, ml1w[...]) + ml1b[...] + res


def _mlp_call(xp, c1, z2p, dis, bl0w, bl0b, bl1w, bl1b, brw, brb,
              ml0w, ml0b, ml1w, ml1b, mrw, mrb):
    full = lambda shape: pl.BlockSpec(shape, lambda i: tuple(0 for _ in shape))
    return pl.pallas_call(
        _mlp_body,
        grid=(GRID,),
        in_specs=[
            pl.BlockSpec((BLK, D), lambda i: (i, 0)),
            pl.BlockSpec((BLK, D), lambda i: (i, 0)),
            pl.BlockSpec((NC, BLK, D), lambda i: (0, i, 0)),
            pl.BlockSpec((BLK, D), lambda i: (i, 0)),
            full(bl0w.shape), full(bl0b.shape),
            full(bl1w.shape), full(bl1b.shape),
            full(brw.shape), full(brb.shape),
            full(ml0w.shape), full(ml0b.shape),
            full(ml1w.shape), full(ml1b.shape),
            full(mrw.shape), full(mrb.shape),
        ],
        out_specs=pl.BlockSpec((BLK, D), lambda i: (i, 0)),
        out_shape=jax.ShapeDtypeStruct((NP, D), jnp.float32),
    )(xp, c1, z2p, dis, bl0w, bl0b, bl1w, bl1b, brw, brb,
      ml0w, ml0b, ml1w, ml1b, mrw, mrb)


def kernel(x, edge_index, b_l0_w, b_l0_b, b_l1_w, b_l1_b, b_res_w, b_res_b,
           m_l0_w, m_l0_b, m_l1_w, m_l1_b, m_res_w, m_res_b):
    row = edge_index[0]
    col = edge_index[1]
    pad = EPAD - E
    pad_dst = DUMMY + jnp.arange(pad, dtype=jnp.int32) % (NP - N)
    colp16 = jnp.concatenate([col, pad_dst]).reshape(NW, CPT // 16, 16)
    pad_h = E0 + E1 - E
    pad_dst_h = DUMMY + jnp.arange(pad_h, dtype=jnp.int32) % (NACC - N)
    row_h = jnp.concatenate([row, jnp.zeros((pad_h,), jnp.int32)])
    col_h = jnp.concatenate([col, pad_dst_h])
    row0 = row_h[:E0].reshape(NS, M0 * CH)
    col0 = col_h[:E0].reshape(NS, M0, CH)
    row1 = row_h[E0:].reshape(NS, M1 * CH)
    col1 = col_h[E0:].reshape(NS, M1, CH)
    xp = jnp.pad(x, ((0, NP - N), (0, 0)))
    zeros128 = jnp.zeros((128, D), jnp.float32)
    m_l0_b = m_l0_b.reshape(1, -1)
    m_l1_b = m_l1_b.reshape(1, -1)
    m_res_b = m_res_b.reshape(1, -1)

    degp = _run_deg(colp16)
    degp3 = degp.reshape(NW, GRID, BLK // 128, 128)

    y1, dis = _scale_call(degp3, xp)
    z1p = _run_hop(y1, row0, col0, row1, col1, zeros128)
    c1, y2 = _mid_call(z1p, dis)
    z2p = _run_hop(y2, row0, col0, row1, col1, zeros128)
    out = _mlp_call(xp, c1, z2p, dis,
                    b_l0_w, b_l0_b, b_l1_w, b_l1_b, b_res_w, b_res_b,
                    m_l0_w, m_l0_b, m_l1_w, m_l1_b, m_res_w, m_res_b)
    return out[:N]
